# baseline (speedup 1.0000x reference)
import functools

import jax
import jax.numpy as jnp
from jax import lax
from jax.experimental import pallas as pl
from jax.experimental.pallas import tpu as pltpu
from jax.experimental.pallas import tpu_sc as plsc

H = 128
NREL = 16
NC = 2
NS = 16
NW = NC * NS
C = 128


BCH = 8
BE = BCH * C


def _sc_agg_build(n_pad, cpw):
    epw = cpw * C
    nblk = cpw // BCH
    rows_per_tile = n_pad // NS
    mesh = plsc.VectorSubcoreMesh(core_axis_name="c", subcore_axis_name="s")

    @functools.partial(
        pl.kernel,
        out_type=jax.ShapeDtypeStruct((NC, n_pad, H), jnp.float32),
        mesh=mesh,
        compiler_params=pltpu.CompilerParams(needs_layout_passes=False),
        scratch_types=[
            pltpu.VMEM((2, BE), jnp.int32),
            pltpu.VMEM((2, BCH, C), jnp.int32),
            pltpu.VMEM((2, BE), jnp.int32),
            pltpu.VMEM((2, BE), jnp.float32),
            pltpu.VMEM((2, C, H), jnp.float32),
            pltpu.VMEM((NREL, H), jnp.float32),
            pltpu.VMEM_SHARED((n_pad, H), jnp.float32),
            pltpu.SemaphoreType.DMA,
            pltpu.SemaphoreType.DMA,
            pltpu.SemaphoreType.DMA,
        ],
    )
    def sc_agg(h_hbm, src_hbm, dst_hbm, et_hbm, nr_hbm, w_hbm, out_hbm,
               src_v, dst_v, et_v, nr_v, rows_v, w_v, agg_sh,
               msem, gsem, ssem):
        cid = lax.axis_index("c")
        sid = lax.axis_index("s")
        wid = cid * NS + sid

        def md_copies(b, slot):
            ebase = wid * epw + b * BE
            return (
                (src_hbm.at[pl.ds(ebase, BE)], src_v.at[slot]),
                (dst_hbm.at[wid, pl.ds(b * BCH, BCH)], dst_v.at[slot]),
                (et_hbm.at[pl.ds(ebase, BE)], et_v.at[slot]),
                (nr_hbm.at[pl.ds(ebase, BE)], nr_v.at[slot]),
            )

        def md_issue(b, slot):
            for s_ref, d_ref in md_copies(b, slot):
                pltpu.async_copy(s_ref, d_ref, msem)

        def md_wait(b, slot):
            for s_ref, d_ref in md_copies(b, slot):
                pltpu.make_async_copy(s_ref, d_ref, msem).wait()

        def gather_issue(slot, mslot, kk):
            pltpu.async_copy(
                h_hbm.at[src_v.at[mslot, pl.ds(kk * C, C)]],
                rows_v.at[slot], gsem)

        def gather_wait(slot):
            pltpu.make_async_copy(
                h_hbm.at[src_v.at[0, pl.ds(0, C)]],
                rows_v.at[slot], gsem).wait()

        def scatter_issue(slot, mslot, kk):
            pltpu.async_copy(
                rows_v.at[slot], agg_sh.at[dst_v.at[mslot, kk]],
                ssem, add=True)

        def scatter_wait(slot):
            pltpu.make_async_copy(
                rows_v.at[slot], agg_sh.at[dst_v.at[0, 0]], ssem).wait()

        md_issue(0, 0)
        pltpu.sync_copy(w_hbm, w_v)

        zero = jnp.zeros((16,), jnp.float32)

        def zrow(i, _):
            for s in range(8):
                rows_v[0, i, pl.ds(16 * s, 16)] = zero
            return 0

        lax.fori_loop(0, C, zrow, 0)
        for k in range(rows_per_tile // C):
            pltpu.sync_copy(
                rows_v.at[0],
                agg_sh.at[pl.ds(sid * rows_per_tile + k * C, C)])
        plsc.subcore_barrier()

        lanes = [lax.iota(jnp.int32, 16) + 16 * s for s in range(8)]

        def compute(slot, mslot, kk):
            @plsc.parallel_loop(0, C, unroll=4)
            def edge(cc):
                esplat = jnp.full((16,), kk * C, jnp.int32) + cc
                msplat = jnp.full((16,), 0, jnp.int32) + mslot
                et_b = plsc.load_gather(et_v, [msplat, esplat])
                nb = plsc.load_gather(nr_v, [msplat, esplat])
                for s in range(8):
                    w = plsc.load_gather(w_v, [et_b, lanes[s]])
                    r = rows_v[slot, cc, pl.ds(16 * s, 16)]
                    rows_v[slot, cc, pl.ds(16 * s, 16)] = r * (w * nb)

        md_wait(0, 0)
        gather_issue(0, 0, 0)

        def block(b, _):
            pb = lax.rem(b, 2)
            pbn = lax.rem(b + 1, 2)
            for k in range(BCH):
                p = k % 2
                q = 1 - p
                if k == 0:
                    @pl.when(b + 1 < nblk)
                    def _():
                        md_issue(b + 1, pbn)

                gather_wait(p)
                @pl.when(jnp.logical_or(b > 0, k > 0))
                def _():
                    scatter_wait(q)

                if k == BCH - 1:
                    @pl.when(b + 1 < nblk)
                    def _():
                        md_wait(b + 1, pbn)
                        gather_issue(q, pbn, 0)
                else:
                    gather_issue(q, pb, k + 1)
                compute(p, pb, k)
                scatter_issue(p, pb, k)
            return 0

        lax.fori_loop(0, nblk, block, 0)
        scatter_wait((BCH - 1) % 2)
        plsc.subcore_barrier()
        pltpu.sync_copy(
            agg_sh.at[pl.ds(sid * rows_per_tile, rows_per_tile)],
            out_hbm.at[cid, pl.ds(sid * rows_per_tile, rows_per_tile)])

    return sc_agg


def _self_loop(h, loop_w, b2d):
    n, _ = h.shape
    br = 1000

    def body(h_ref, w_ref, b_ref, o_ref):
        o_ref[...] = jnp.dot(h_ref[...], w_ref[...],
                             preferred_element_type=jnp.float32) + b_ref[...]

    return pl.pallas_call(
        body,
        grid=(n // br,),
        in_specs=[
            pl.BlockSpec((br, H), lambda i: (i, 0)),
            pl.BlockSpec((H, H), lambda i: (0, 0)),
            pl.BlockSpec((1, H), lambda i: (0, 0)),
        ],
        out_specs=pl.BlockSpec((br, H), lambda i: (i, 0)),
        out_shape=jax.ShapeDtypeStruct((n, H), jnp.float32),
    )(h, loop_w, b2d)


def _combine(parts, mm, relu):
    n, _ = mm.shape
    br = 1000

    def body(p_ref, m_ref, o_ref):
        acc = p_ref[0] + p_ref[1] + m_ref[...]
        o_ref[...] = jnp.maximum(acc, 0.0) if relu else acc

    return pl.pallas_call(
        body,
        grid=(n // br,),
        in_specs=[
            pl.BlockSpec((NC, br, H), lambda i: (0, i, 0)),
            pl.BlockSpec((br, H), lambda i: (i, 0)),
        ],
        out_specs=pl.BlockSpec((br, H), lambda i: (i, 0)),
        out_shape=jax.ShapeDtypeStruct((n, H), jnp.float32),
    )(parts, mm)


def kernel(x, edge_index, etype, norm, W1, loop1, b1, W2, loop2, b2):
    n_nodes = x.shape[0]
    e = edge_index.shape[1]
    chunks = -(-e // C)
    cpw = -(-chunks // (NW * BCH)) * BCH
    e_pad = NW * cpw * C

    src = edge_index[0].astype(jnp.int32)
    dst = edge_index[1].astype(jnp.int32)
    et = etype.astype(jnp.int32)
    nr = norm.reshape(-1)
    pad = e_pad - e
    src = jnp.pad(src, (0, pad))
    dst = jnp.pad(dst, (0, pad)).reshape(NW, cpw, C)
    et = jnp.pad(et, (0, pad))
    nr = jnp.pad(nr, (0, pad))

    n_pad = -(-n_nodes // (NS * C)) * (NS * C)
    sc_agg = _sc_agg_build(n_pad, cpw)
    p1 = sc_agg(x, src, dst, et, nr, W1)
    mm1 = _self_loop(x, loop1, b1.reshape(1, H))
    h1 = _combine(p1, mm1, relu=True)
    p2 = sc_agg(h1, src, dst, et, nr, W2)
    mm2 = _self_loop(h1, loop2, b2.reshape(1, H))
    return _combine(p2, mm2, relu=False)


# speedup vs baseline: 1.1689x; 1.1689x over previous
import functools

import jax
import jax.numpy as jnp
from jax import lax
from jax.experimental import pallas as pl
from jax.experimental.pallas import tpu as pltpu
from jax.experimental.pallas import tpu_sc as plsc

H = 128
NREL = 16
NC = 2
NS = 16
NW = NC * NS
C = 128


BCH = 8
BE = BCH * C


def _sc_agg_build(n_pad, cpw):
    epw = cpw * C
    nblk = cpw // BCH
    rows_per_tile = n_pad // NS
    mesh = plsc.VectorSubcoreMesh(core_axis_name="c", subcore_axis_name="s")

    @functools.partial(
        pl.kernel,
        out_type=jax.ShapeDtypeStruct((NC, n_pad, H), jnp.float32),
        mesh=mesh,
        compiler_params=pltpu.CompilerParams(needs_layout_passes=False),
        scratch_types=[
            pltpu.VMEM((2, BE), jnp.int32),
            pltpu.VMEM((2, BCH, C), jnp.int32),
            pltpu.VMEM((2, BE), jnp.int32),
            pltpu.VMEM((2, BE), jnp.float32),
            pltpu.VMEM((2, C, H), jnp.float32),
            pltpu.VMEM((NREL, H), jnp.float32),
            pltpu.VMEM_SHARED((n_pad, H), jnp.float32),
            pltpu.SemaphoreType.DMA,
            pltpu.SemaphoreType.DMA,
            pltpu.SemaphoreType.DMA,
        ],
    )
    def sc_agg(h_hbm, src_hbm, dst_hbm, et_hbm, nr_hbm, w_hbm, out_hbm,
               src_v, dst_v, et_v, nr_v, rows_v, w_v, agg_sh,
               msem, gsem, ssem):
        cid = lax.axis_index("c")
        sid = lax.axis_index("s")
        wid = cid * NS + sid

        def md_copies(b, slot):
            ebase = wid * epw + b * BE
            return (
                (src_hbm.at[pl.ds(ebase, BE)], src_v.at[slot]),
                (dst_hbm.at[wid, pl.ds(b * BCH, BCH)], dst_v.at[slot]),
                (et_hbm.at[pl.ds(ebase, BE)], et_v.at[slot]),
                (nr_hbm.at[pl.ds(ebase, BE)], nr_v.at[slot]),
            )

        def md_issue(b, slot):
            for s_ref, d_ref in md_copies(b, slot):
                pltpu.async_copy(s_ref, d_ref, msem)

        def md_wait(b, slot):
            for s_ref, d_ref in md_copies(b, slot):
                pltpu.make_async_copy(s_ref, d_ref, msem).wait()

        def gather_issue(slot, mslot, kk):
            pltpu.async_copy(
                h_hbm.at[src_v.at[mslot, pl.ds(kk * C, C)]],
                rows_v.at[slot], gsem)

        def gather_wait(slot):
            pltpu.make_async_copy(
                h_hbm.at[src_v.at[0, pl.ds(0, C)]],
                rows_v.at[slot], gsem).wait()

        def scatter_issue(slot, mslot, kk):
            pltpu.async_copy(
                rows_v.at[slot], agg_sh.at[dst_v.at[mslot, kk]],
                ssem, add=True)

        def scatter_wait(slot):
            pltpu.make_async_copy(
                rows_v.at[slot], agg_sh.at[dst_v.at[0, 0]], ssem).wait()

        md_issue(0, 0)
        pltpu.sync_copy(w_hbm, w_v)

        zero = jnp.zeros((16,), jnp.float32)

        def zrow(i, _):
            for s in range(8):
                rows_v[0, i, pl.ds(16 * s, 16)] = zero
            return 0

        lax.fori_loop(0, C, zrow, 0)
        for k in range(rows_per_tile // C):
            pltpu.sync_copy(
                rows_v.at[0],
                agg_sh.at[pl.ds(sid * rows_per_tile + k * C, C)])
        plsc.subcore_barrier()

        lanes = [lax.iota(jnp.int32, 16) + 16 * s for s in range(8)]

        def compute(slot, mslot, kk):
            @plsc.parallel_loop(0, C, unroll=4)
            def edge(cc):
                esplat = jnp.full((16,), kk * C, jnp.int32) + cc
                msplat = jnp.full((16,), 0, jnp.int32) + mslot
                et_b = plsc.load_gather(et_v, [msplat, esplat])
                nb = plsc.load_gather(nr_v, [msplat, esplat])
                for s in range(8):
                    w = plsc.load_gather(w_v, [et_b, lanes[s]])
                    r = rows_v[slot, cc, pl.ds(16 * s, 16)]
                    rows_v[slot, cc, pl.ds(16 * s, 16)] = r * (w * nb)

        md_wait(0, 0)
        gather_issue(0, 0, 0)

        def block(b, _):
            pb = lax.rem(b, 2)
            pbn = lax.rem(b + 1, 2)
            for k in range(BCH):
                p = k % 2
                q = 1 - p
                if k == 0:
                    @pl.when(b + 1 < nblk)
                    def _():
                        md_issue(b + 1, pbn)

                gather_wait(p)
                @pl.when(jnp.logical_or(b > 0, k > 0))
                def _():
                    scatter_wait(q)

                if k == BCH - 1:
                    @pl.when(b + 1 < nblk)
                    def _():
                        md_wait(b + 1, pbn)
                        gather_issue(q, pbn, 0)
                else:
                    gather_issue(q, pb, k + 1)
                compute(p, pb, k)
                scatter_issue(p, pb, k)
            return 0

        lax.fori_loop(0, nblk, block, 0)
        scatter_wait((BCH - 1) % 2)
        plsc.subcore_barrier()
        pltpu.sync_copy(
            agg_sh.at[pl.ds(sid * rows_per_tile, rows_per_tile)],
            out_hbm.at[cid, pl.ds(sid * rows_per_tile, rows_per_tile)])

    return sc_agg


def _combine(parts, h, loop_w, b2d, relu):
    n, _ = h.shape
    br = 1000

    def body(p_ref, h_ref, w_ref, b_ref, o_ref):
        acc = p_ref[0] + p_ref[1] + b_ref[...]
        acc += jnp.dot(h_ref[...], w_ref[...],
                       preferred_element_type=jnp.float32)
        o_ref[...] = jnp.maximum(acc, 0.0) if relu else acc

    return pl.pallas_call(
        body,
        grid=(n // br,),
        in_specs=[
            pl.BlockSpec((NC, br, H), lambda i: (0, i, 0)),
            pl.BlockSpec((br, H), lambda i: (i, 0)),
            pl.BlockSpec((H, H), lambda i: (0, 0)),
            pl.BlockSpec((1, H), lambda i: (0, 0)),
        ],
        out_specs=pl.BlockSpec((br, H), lambda i: (i, 0)),
        out_shape=jax.ShapeDtypeStruct((n, H), jnp.float32),
    )(parts, h, loop_w, b2d)


def kernel(x, edge_index, etype, norm, W1, loop1, b1, W2, loop2, b2):
    n_nodes = x.shape[0]
    e = edge_index.shape[1]
    chunks = -(-e // C)
    cpw = -(-chunks // (NW * BCH)) * BCH
    e_pad = NW * cpw * C

    src = edge_index[0].astype(jnp.int32)
    dst = edge_index[1].astype(jnp.int32)
    et = etype.astype(jnp.int32)
    nr = norm.reshape(-1)
    pad = e_pad - e
    src = jnp.pad(src, (0, pad))
    dst = jnp.pad(dst, (0, pad)).reshape(NW, cpw, C)
    et = jnp.pad(et, (0, pad))
    nr = jnp.pad(nr, (0, pad))

    n_pad = -(-n_nodes // (NS * C)) * (NS * C)
    sc_agg = _sc_agg_build(n_pad, cpw)
    p1 = sc_agg(x, src, dst, et, nr, W1)
    h1 = _combine(p1, x, loop1, b1.reshape(1, H), relu=True)
    p2 = sc_agg(h1, src, dst, et, nr, W2)
    return _combine(p2, h1, loop2, b2.reshape(1, H), relu=False)
